# Initial kernel scaffold; baseline (speedup 1.0000x reference)
import jax
import jax.numpy as jnp
from jax.experimental import pallas as pl


def kernel(surface_x, graph_x, rbf_weights, W_s_pre, b_s_pre, W_g_pre, b_g_pre, W_s_post, b_s_post, W_g_post, b_g_post):
    raise NotImplementedError("write your pallas kernel here")


# speedup vs baseline: 1.1920x; 1.1920x over previous
import functools

import jax
import jax.numpy as jnp
from jax.experimental import pallas as pl
from jax.experimental.pallas import tpu as pltpu

Ns, Ng, D = 8192, 4096, 256
TI, TJ = 1024, 1024
NI, NJ = Ns // TI, Ng // TJ

_F32 = jnp.float32
_BF16 = jnp.bfloat16


def _dot(a, b):
    return jnp.dot(a, b, preferred_element_type=_F32)


def _body(sx_ref, gx_ref, rbf_ref,
          wsp_ref, bsp_ref, wgp_ref, bgp_ref,
          wsa_ref, wsb_ref, bspo_ref, wga_ref, wgb_ref, bgpo_ref,
          xs_ref, xg_ref,
          xs_pre, xs_preT, xg_pre, xg_accT):
    i = pl.program_id(0)
    j = pl.program_id(1)

    @pl.when((i == 0) & (j == 0))
    def _pre_mlps():
        s = jnp.maximum(_dot(sx_ref[...], wsp_ref[...]) + bsp_ref[...], 0.0)
        s16 = s.astype(_BF16)
        xs_pre[...] = s16
        xs_preT[...] = s16.T
        g = jnp.maximum(_dot(gx_ref[...], wgp_ref[...]) + bgp_ref[...], 0.0)
        xg_pre[...] = g.astype(_BF16)

    a = rbf_ref[...].astype(_BF16)

    contrib_s = _dot(a, xg_pre[pl.ds(j * TJ, TJ), :])

    @pl.when(j == 0)
    def _():
        xs_ref[pl.ds(i * TI, TI), :] = contrib_s

    @pl.when(j > 0)
    def _():
        xs_ref[pl.ds(i * TI, TI), :] += contrib_s

    contrib_g = _dot(xs_preT[:, pl.ds(i * TI, TI)], a)

    @pl.when(i == 0)
    def _():
        xg_accT[:, pl.ds(j * TJ, TJ)] = contrib_g

    @pl.when(i > 0)
    def _():
        xg_accT[:, pl.ds(j * TJ, TJ)] += contrib_g

    @pl.when(j == pl.num_programs(1) - 1)
    def _post_s():
        acc = xs_ref[pl.ds(i * TI, TI), :].astype(_BF16)
        pre = xs_pre[pl.ds(i * TI, TI), :]
        r = _dot(pre, wsa_ref[...]) + _dot(acc, wsb_ref[...]) + bspo_ref[...]
        xs_ref[pl.ds(i * TI, TI), :] = jnp.maximum(r, 0.0)

    @pl.when(i == pl.num_programs(0) - 1)
    def _post_g():
        acc = xg_accT[:, pl.ds(j * TJ, TJ)].astype(_BF16).T
        pre = xg_pre[pl.ds(j * TJ, TJ), :]
        r = _dot(pre, wga_ref[...]) + _dot(acc, wgb_ref[...]) + bgpo_ref[...]
        xg_ref[pl.ds(j * TJ, TJ), :] = jnp.maximum(r, 0.0)


@functools.partial(jax.jit, static_argnames=("interpret",))
def _run(sx, gx, rbf, wsp, bsp, wgp, bgp, wsa, wsb, bspo, wga, wgb, bgpo,
         interpret=False):
    whole = lambda shape: pl.BlockSpec(shape, lambda i, j: (0, 0))
    return pl.pallas_call(
        _body,
        grid=(NI, NJ),
        in_specs=[
            whole((Ns, D)),
            whole((Ng, D)),
            pl.BlockSpec((TI, TJ), lambda i, j: (i, j)),
            whole((D, D)), whole((1, D)),
            whole((D, D)), whole((1, D)),
            whole((D, D)), whole((D, D)), whole((1, D)),
            whole((D, D)), whole((D, D)), whole((1, D)),
        ],
        out_specs=[whole((Ns, D)), whole((Ng, D))],
        out_shape=[
            jax.ShapeDtypeStruct((Ns, D), _F32),
            jax.ShapeDtypeStruct((Ng, D), _F32),
        ],
        scratch_shapes=[
            pltpu.VMEM((Ns, D), _BF16),
            pltpu.VMEM((D, Ns), _BF16),
            pltpu.VMEM((Ng, D), _BF16),
            pltpu.VMEM((D, Ng), _F32),
        ],
        compiler_params=pltpu.CompilerParams(
            dimension_semantics=("arbitrary", "arbitrary"),
        ),
        interpret=interpret,
    )(sx, gx, rbf, wsp, bsp, wgp, bgp, wsa, wsb, bspo, wga, wgb, bgpo)


def kernel(surface_x, graph_x, rbf_weights,
           W_s_pre, b_s_pre, W_g_pre, b_g_pre,
           W_s_post, b_s_post, W_g_post, b_g_post):
    sx = surface_x.astype(_BF16)
    gx = graph_x.astype(_BF16)
    wsp = W_s_pre.astype(_BF16)
    wgp = W_g_pre.astype(_BF16)
    wsa = W_s_post[:D].astype(_BF16)
    wsb = W_s_post[D:].astype(_BF16)
    wga = W_g_post[:D].astype(_BF16)
    wgb = W_g_post[D:].astype(_BF16)
    bsp = b_s_pre.reshape(1, D)
    bgp = b_g_pre.reshape(1, D)
    bspo = b_s_post.reshape(1, D)
    bgpo = b_g_post.reshape(1, D)
    xs, xg = _run(sx, gx, rbf_weights, wsp, bsp, wgp, bgp,
                  wsa, wsb, bspo, wga, wgb, bgpo)
    return (xs, xg)


# speedup vs baseline: 1.2741x; 1.0689x over previous
import functools

import jax
import jax.numpy as jnp
from jax.experimental import pallas as pl
from jax.experimental.pallas import tpu as pltpu

Ns, Ng, D = 8192, 4096, 256
TI, TJ = 2048, 1024
NI, NJ = Ns // TI, Ng // TJ

_F32 = jnp.float32
_BF16 = jnp.bfloat16


def _dot(a, b):
    return jnp.dot(a, b, preferred_element_type=_F32)


def _body(sx_ref, gx_ref, rbf_ref,
          wsp_ref, bsp_ref, wgp_ref, bgp_ref,
          wsa_ref, wsb_ref, bspo_ref, wga_ref, wgb_ref, bgpo_ref,
          xs_ref, xg_ref,
          xs_pre, xs_preT, xg_pre, xg_accT):
    i = pl.program_id(0)
    j = pl.program_id(1)

    @pl.when((i == 0) & (j == 0))
    def _pre_mlps():
        s = jnp.maximum(_dot(sx_ref[...], wsp_ref[...]) + bsp_ref[...], 0.0)
        s16 = s.astype(_BF16)
        xs_pre[...] = s16
        xs_preT[...] = s16.T
        g = jnp.maximum(_dot(gx_ref[...], wgp_ref[...]) + bgp_ref[...], 0.0)
        xg_pre[...] = g.astype(_BF16)

    a = rbf_ref[...].astype(_BF16)

    contrib_s = _dot(a, xg_pre[pl.ds(j * TJ, TJ), :])

    @pl.when(j == 0)
    def _():
        xs_ref[pl.ds(i * TI, TI), :] = contrib_s

    @pl.when(j > 0)
    def _():
        xs_ref[pl.ds(i * TI, TI), :] += contrib_s

    contrib_g = _dot(xs_preT[:, pl.ds(i * TI, TI)], a)

    @pl.when(i == 0)
    def _():
        xg_accT[:, pl.ds(j * TJ, TJ)] = contrib_g

    @pl.when(i > 0)
    def _():
        xg_accT[:, pl.ds(j * TJ, TJ)] += contrib_g

    @pl.when(j == pl.num_programs(1) - 1)
    def _post_s():
        acc = xs_ref[pl.ds(i * TI, TI), :].astype(_BF16)
        pre = xs_pre[pl.ds(i * TI, TI), :]
        r = _dot(pre, wsa_ref[...]) + _dot(acc, wsb_ref[...]) + bspo_ref[...]
        xs_ref[pl.ds(i * TI, TI), :] = jnp.maximum(r, 0.0)

    @pl.when(i == pl.num_programs(0) - 1)
    def _post_g():
        acc = xg_accT[:, pl.ds(j * TJ, TJ)].astype(_BF16).T
        pre = xg_pre[pl.ds(j * TJ, TJ), :]
        r = _dot(pre, wga_ref[...]) + _dot(acc, wgb_ref[...]) + bgpo_ref[...]
        xg_ref[pl.ds(j * TJ, TJ), :] = jnp.maximum(r, 0.0)


@functools.partial(jax.jit, static_argnames=("interpret",))
def _run(sx, gx, rbf, wsp, bsp, wgp, bgp, wsa, wsb, bspo, wga, wgb, bgpo,
         interpret=False):
    whole = lambda shape: pl.BlockSpec(shape, lambda i, j: (0, 0))
    return pl.pallas_call(
        _body,
        grid=(NI, NJ),
        in_specs=[
            whole((Ns, D)),
            whole((Ng, D)),
            pl.BlockSpec((TI, TJ), lambda i, j: (i, j)),
            whole((D, D)), whole((1, D)),
            whole((D, D)), whole((1, D)),
            whole((D, D)), whole((D, D)), whole((1, D)),
            whole((D, D)), whole((D, D)), whole((1, D)),
        ],
        out_specs=[whole((Ns, D)), whole((Ng, D))],
        out_shape=[
            jax.ShapeDtypeStruct((Ns, D), _F32),
            jax.ShapeDtypeStruct((Ng, D), _F32),
        ],
        scratch_shapes=[
            pltpu.VMEM((Ns, D), _BF16),
            pltpu.VMEM((D, Ns), _BF16),
            pltpu.VMEM((Ng, D), _BF16),
            pltpu.VMEM((D, Ng), _F32),
        ],
        compiler_params=pltpu.CompilerParams(
            dimension_semantics=("arbitrary", "arbitrary"),
        ),
        interpret=interpret,
    )(sx, gx, rbf, wsp, bsp, wgp, bgp, wsa, wsb, bspo, wga, wgb, bgpo)


def kernel(surface_x, graph_x, rbf_weights,
           W_s_pre, b_s_pre, W_g_pre, b_g_pre,
           W_s_post, b_s_post, W_g_post, b_g_post):
    sx = surface_x.astype(_BF16)
    gx = graph_x.astype(_BF16)
    wsp = W_s_pre.astype(_BF16)
    wgp = W_g_pre.astype(_BF16)
    wsa = W_s_post[:D].astype(_BF16)
    wsb = W_s_post[D:].astype(_BF16)
    wga = W_g_post[:D].astype(_BF16)
    wgb = W_g_post[D:].astype(_BF16)
    bsp = b_s_pre.reshape(1, D)
    bgp = b_g_pre.reshape(1, D)
    bspo = b_s_post.reshape(1, D)
    bgpo = b_g_post.reshape(1, D)
    xs, xg = _run(sx, gx, rbf_weights, wsp, bsp, wgp, bgp,
                  wsa, wsb, bspo, wga, wgb, bgpo)
    return (xs, xg)
